# baseline (speedup 1.0000x reference)
import functools

import jax
import jax.numpy as jnp
from jax.experimental import pallas as pl

N = 10000
DEG = 16
E = N * DEG
D = 256
NUM_EXPERTS = 8
TOP_K = 2
HID = 512
COEF = 0.01
K_LIST = jnp.array([0.125, 0.25, 0.375, 0.5, 0.625, 0.75, 0.875, 1.0],
                   dtype=jnp.float32)

_BE = 1024


def _mlp_block(feats_ref, gates_ref, w1_ref, b1_ref, w2_ref, b2_ref, out_ref):
    feats = feats_ref[...]
    acc = jnp.zeros((_BE,), dtype=jnp.float32)
    for e in range(NUM_EXPERTS):
        h = jnp.maximum(
            jnp.dot(feats, w1_ref[e], preferred_element_type=jnp.float32)
            + b1_ref[e][None, :], 0.0)
        s = jnp.dot(h, w2_ref[e], preferred_element_type=jnp.float32)[:, 0]
        s = s + b2_ref[e][0]
        acc = acc + gates_ref[:, e] * jax.nn.sigmoid(s)
    out_ref[...] = acc / NUM_EXPERTS


def _expert_gated(feats, gates_e, W1, b1, W2, b2):
    grid = ((E + _BE - 1) // _BE,)
    return pl.pallas_call(
        _mlp_block,
        grid=grid,
        in_specs=[
            pl.BlockSpec((_BE, 2 * D + 1), lambda i: (i, 0)),
            pl.BlockSpec((_BE, NUM_EXPERTS), lambda i: (i, 0)),
            pl.BlockSpec((NUM_EXPERTS, 2 * D + 1, HID), lambda i: (0, 0, 0)),
            pl.BlockSpec((NUM_EXPERTS, HID), lambda i: (0, 0)),
            pl.BlockSpec((NUM_EXPERTS, HID, 1), lambda i: (0, 0, 0)),
            pl.BlockSpec((NUM_EXPERTS, 1), lambda i: (0, 0)),
        ],
        out_specs=pl.BlockSpec((_BE,), lambda i: (i,)),
        out_shape=jax.ShapeDtypeStruct((E,), jnp.float32),
    )(feats, gates_e, W1, b1, W2, b2)


def _binary_step(x):
    return (x > 0.0).astype(jnp.float32)


def _cv_squared(v):
    v = v.astype(jnp.float32)
    return jnp.var(v, ddof=1) / (jnp.mean(v) ** 2 + 1e-10)


def kernel(input_nodes, adjacency_index, adjacency_attr, temperature,
           data_shape, w_gate, W1, b1, W2, b2):
    src = adjacency_index[0]
    dst = adjacency_index[1]
    clean_logits = input_nodes @ w_gate
    top_logits, top_indices = jax.lax.top_k(clean_logits,
                                            min(TOP_K + 1, NUM_EXPERTS))
    top_k_logits = top_logits[:, :TOP_K]
    top_k_indices = top_indices[:, :TOP_K]
    top_k_gates = jax.nn.softmax(top_k_logits, axis=1)
    gates = jnp.zeros_like(clean_logits).at[
        jnp.arange(N)[:, None], top_k_indices].set(top_k_gates)
    load = (gates > 0).sum(axis=0)
    relevance = gates.sum(axis=0)
    loss_val = COEF * (_cv_squared(relevance) + _cv_squared(load))

    edge_selections = gates[src]
    feats = jnp.concatenate(
        [input_nodes[src], input_nodes[dst], adjacency_attr], axis=1)
    gated_output = _expert_gated(feats, edge_selections, W1, b1, W2, b2)

    edges_per_node = jnp.bincount(src, length=N)
    selected_k_per_node = (gates * K_LIST[None, :]).sum(axis=1)
    edges_selected = selected_k_per_node * edges_per_node
    edges_selected_l = jnp.round(edges_selected).astype(jnp.int32)
    edges_selected_l = jnp.where(edges_selected_l > 0, edges_selected_l, 1)
    sorted_idx = jnp.argsort(-gated_output)
    sorted_vals = gated_output[sorted_idx]
    sparse_idx0 = src[sorted_idx]
    sorted_idx_reorder = jnp.argsort(sparse_idx0, stable=True)
    reordered_scores = sorted_vals[sorted_idx_reorder]
    start = jnp.concatenate([jnp.zeros((1,), dtype=edges_per_node.dtype),
                             jnp.cumsum(edges_per_node[:-1])])
    end = jnp.abs(start + edges_selected_l - 1).astype(jnp.int32)
    node_thresholds = reordered_scores[end]
    augmented = jnp.repeat(node_thresholds, edges_per_node,
                           total_repeat_length=E)
    mask = _binary_step(reordered_scores - augmented + 1e-12)
    final_sort_idx = jnp.argsort(sorted_idx_reorder)
    mask = mask[final_sort_idx]
    mask = mask[sorted_idx]
    return mask, loss_val


# speedup vs baseline: 1.6192x; 1.6192x over previous
import jax
import jax.numpy as jnp
import numpy as np
from jax.experimental import pallas as pl
from jax.experimental.pallas import tpu as pltpu

N = 10000
DEG = 16
E = N * DEG
D = 256
NUM_EXPERTS = 8
HID = 512
FAN_IN = 2 * D + 1
COEF = 0.01
_K_LIST = np.array([0.125, 0.25, 0.375, 0.5, 0.625, 0.75, 0.875, 1.0],
                   dtype=np.float32)

NB = 64
RB = NB * DEG
NCHUNK = 213


def _moe_block(e0_ref, e1_ref, xsrc_ref, xdst_ref, attr_ref, g_ref,
               w1_ref, b1_ref, w2_ref, b2_ref, po0_ref, po1_ref):
    i = pl.program_id(0)
    xsrc = xsrc_ref[...]
    xdst = xdst_ref[...]
    attr = attr_ref[...]
    xsrc_rep = jnp.broadcast_to(xsrc[:, None, :], (NB, DEG, D)).reshape(RB, D)
    feats = jnp.concatenate([xsrc_rep, xdst, attr], axis=1)
    g = g_ref[...]

    def expert_pass(e, g_col, po_ref):
        w1 = w1_ref[e]
        h = jnp.maximum(
            jnp.dot(feats, w1, preferred_element_type=jnp.float32)
            + b1_ref[e][None, :], 0.0)
        s = jnp.dot(h, w2_ref[e], preferred_element_type=jnp.float32)
        s = s + b2_ref[e][None, :]
        o = jax.nn.sigmoid(s)
        grep = jnp.broadcast_to(g_col[:, None, :], (NB, DEG, 1)).reshape(RB, 1)
        po_ref[...] = grep * o

    expert_pass(e0_ref[i], g[:, 0:1], po0_ref)
    expert_pass(e1_ref[i], g[:, 1:2], po1_ref)


def _routed_po(chunk_e0, chunk_e1, xsrc_c, xdst_c, attr_c, g_c,
               W1, b1, W2, b2):
    grid_spec = pltpu.PrefetchScalarGridSpec(
        num_scalar_prefetch=2,
        grid=(NCHUNK,),
        in_specs=[
            pl.BlockSpec((NB, D), lambda i, *_: (i, 0)),
            pl.BlockSpec((RB, D), lambda i, *_: (i, 0)),
            pl.BlockSpec((RB, 1), lambda i, *_: (i, 0)),
            pl.BlockSpec((NB, 2), lambda i, *_: (i, 0)),
            pl.BlockSpec((NUM_EXPERTS, FAN_IN, HID), lambda i, *_: (0, 0, 0)),
            pl.BlockSpec((NUM_EXPERTS, HID), lambda i, *_: (0, 0)),
            pl.BlockSpec((NUM_EXPERTS, HID, 1), lambda i, *_: (0, 0, 0)),
            pl.BlockSpec((NUM_EXPERTS, 1), lambda i, *_: (0, 0)),
        ],
        out_specs=[
            pl.BlockSpec((RB, 1), lambda i, *_: (i, 0)),
            pl.BlockSpec((RB, 1), lambda i, *_: (i, 0)),
        ],
    )
    return pl.pallas_call(
        _moe_block,
        grid_spec=grid_spec,
        out_shape=[
            jax.ShapeDtypeStruct((NCHUNK * RB, 1), jnp.float32),
            jax.ShapeDtypeStruct((NCHUNK * RB, 1), jnp.float32),
        ],
    )(chunk_e0, chunk_e1, xsrc_c, xdst_c, attr_c, g_c, W1, b1, W2, b2)


def _binary_step(x):
    return (x > 0.0).astype(jnp.float32)


def _cv_squared(v):
    v = v.astype(jnp.float32)
    return jnp.var(v, ddof=1) / (jnp.mean(v) ** 2 + 1e-10)


def kernel(input_nodes, adjacency_index, adjacency_attr, temperature,
           data_shape, w_gate, W1, b1, W2, b2):
    src = adjacency_index[0]
    dst = adjacency_index[1]
    X = input_nodes

    clean_logits = X @ w_gate
    top_logits, top_indices = jax.lax.top_k(clean_logits, 3)
    tk_logits = top_logits[:, :2]
    tk_idx = top_indices[:, :2]
    tk_gates = jax.nn.softmax(tk_logits, axis=1)
    gates = jnp.zeros_like(clean_logits).at[
        jnp.arange(N)[:, None], tk_idx].set(tk_gates)
    load = (gates > 0).sum(axis=0)
    relevance = gates.sum(axis=0)
    loss_val = COEF * (_cv_squared(relevance) + _cv_squared(load))

    e0 = tk_idx[:, 0].astype(jnp.int32)
    e1 = tk_idx[:, 1].astype(jnp.int32)
    pair_id = e0 * NUM_EXPERTS + e1
    order = jnp.argsort(pair_id, stable=True).astype(jnp.int32)
    pair_sorted = pair_id[order]
    cnt = jnp.bincount(pair_id, length=NUM_EXPERTS * NUM_EXPERTS)
    chunks_per_pair = (cnt + NB - 1) // NB
    chunk_start = jnp.concatenate(
        [jnp.zeros(1, jnp.int32),
         jnp.cumsum(chunks_per_pair)[:-1].astype(jnp.int32)])
    node_off = jnp.concatenate(
        [jnp.zeros(1, jnp.int32), jnp.cumsum(cnt)[:-1].astype(jnp.int32)])
    r = jnp.arange(N, dtype=jnp.int32)
    within = r - node_off[pair_sorted]
    dest = ((chunk_start[pair_sorted] + within // NB) * NB + within % NB)
    nodes_chunked = jnp.zeros(NCHUNK * NB, jnp.int32).at[dest].set(order)
    chunk_e0 = jnp.zeros(NCHUNK, jnp.int32).at[dest // NB].set(e0[order])
    chunk_e1 = jnp.zeros(NCHUNK, jnp.int32).at[dest // NB].set(e1[order])
    pos = jnp.zeros(N, jnp.int32).at[order].set(dest)

    xsrc_c = X[nodes_chunked]
    eidx = (nodes_chunked[:, None] * DEG
            + jnp.arange(DEG, dtype=jnp.int32)[None, :]).reshape(-1)
    xdst_c = X[dst[eidx]]
    attr_c = adjacency_attr[eidx]
    g_c = tk_gates[nodes_chunked]

    po0, po1 = _routed_po(chunk_e0, chunk_e1, xsrc_c, xdst_c, attr_c, g_c,
                          W1, b1, W2, b2)

    rows = (pos[:, None] * DEG
            + jnp.arange(DEG, dtype=jnp.int32)[None, :]).reshape(-1)
    score_edge = (po0[rows, 0] + po1[rows, 0]) / NUM_EXPERTS

    score_nodes = score_edge.reshape(N, DEG)
    sortv = jnp.sort(score_nodes, axis=1)
    edges_per_node = jnp.bincount(src, length=N)
    selected_k_per_node = (gates * jnp.asarray(_K_LIST)[None, :]).sum(axis=1)
    edges_selected = selected_k_per_node * edges_per_node
    k_l = jnp.round(edges_selected).astype(jnp.int32)
    k_l = jnp.where(k_l > 0, k_l, 1)
    thr = jnp.take_along_axis(sortv, (DEG - k_l)[:, None], axis=1)
    mask_edge = _binary_step(score_nodes - thr + 1e-12).reshape(E)

    p = jnp.argsort(-score_edge)
    mask2 = mask_edge[p]
    return mask2[p], loss_val
